# baseline (speedup 1.0000x reference)
import jax
import jax.numpy as jnp
from jax import lax
from jax.experimental import pallas as pl
from jax.experimental.pallas import tpu as pltpu
from jax.experimental.pallas import tpu_sc as plsc

N = 10000
E = 320000
D = 128
HD = D // 2

NC = 2
NS = 16
EPT = E // NS
CHUNK = 80
NCHUNK = EPT // CHUNK
NBUF = 8
NROUND = -(-NCHUNK // NBUF)
RPS = 624
RPS_LAST = N - (NS - 1) * RPS

_mesh = plsc.VectorSubcoreMesh(
    core_axis_name="c", subcore_axis_name="s", num_cores=NC, num_subcores=NS
)


def _agg_body(xs_hbm, src_hbm, dst_hbm, zero_hbm, out_hbm,
              src_v, dst_v, rows_v, gsem, ssem, acc_sh):
    c = lax.axis_index("c")
    s = lax.axis_index("s")

    @pl.when(s < NS - 1)
    def _():
        pltpu.sync_copy(zero_hbm.at[pl.ds(s * RPS, RPS)],
                        acc_sh.at[pl.ds(s * RPS, RPS)])

    @pl.when(s == NS - 1)
    def _():
        pltpu.sync_copy(zero_hbm.at[pl.ds((NS - 1) * RPS, RPS_LAST)],
                        acc_sh.at[pl.ds((NS - 1) * RPS, RPS_LAST)])

    pltpu.sync_copy(src_hbm.at[s], src_v)
    pltpu.sync_copy(dst_hbm.at[s], dst_v)
    plsc.subcore_barrier()

    xc = xs_hbm.at[c]

    for b in range(NBUF):
        pltpu.async_copy(xc.at[src_v.at[b]], rows_v.at[b], gsem.at[b])

    def round_body(g, cc):
        for b in range(NBUF):
            j = g * NBUF + b

            @pl.when(j < NCHUNK)
            def _():
                pltpu.make_async_copy(xc.at[src_v.at[j]], rows_v.at[b],
                                      gsem.at[b]).wait()
                pltpu.async_copy(rows_v.at[b], acc_sh.at[dst_v.at[j]],
                                 ssem.at[b], add=True)
        for b in range(NBUF):
            jn = (g + 1) * NBUF + b

            @pl.when(jn < NCHUNK)
            def _():
                pltpu.make_async_copy(rows_v.at[b], acc_sh.at[dst_v.at[jn]],
                                      ssem.at[b]).wait()
                pltpu.async_copy(xc.at[src_v.at[jn]], rows_v.at[b],
                                 gsem.at[b])
        return cc

    lax.fori_loop(0, NROUND, round_body, 0)
    for b in range(NBUF):
        pltpu.make_async_copy(rows_v.at[b], acc_sh.at[dst_v.at[b]],
                              ssem.at[b]).wait()
    plsc.subcore_barrier()

    @pl.when(s < NS - 1)
    def _():
        pltpu.sync_copy(acc_sh.at[pl.ds(s * RPS, RPS)],
                        out_hbm.at[pl.ds(s * RPS, RPS), pl.ds(c * HD, HD)])

    @pl.when(s == NS - 1)
    def _():
        pltpu.sync_copy(acc_sh.at[pl.ds((NS - 1) * RPS, RPS_LAST)],
                        out_hbm.at[pl.ds((NS - 1) * RPS, RPS_LAST),
                                   pl.ds(c * HD, HD)])


_agg = pl.kernel(
    _agg_body,
    out_type=jax.ShapeDtypeStruct((N, D), jnp.float32),
    mesh=_mesh,
    scratch_types=[
        pltpu.VMEM((NCHUNK, CHUNK), jnp.int32),
        pltpu.VMEM((NCHUNK, CHUNK), jnp.int32),
        pltpu.VMEM((NBUF, CHUNK, HD), jnp.float32),
        pltpu.SemaphoreType.DMA((NBUF,)),
        pltpu.SemaphoreType.DMA((NBUF,)),
        pltpu.VMEM_SHARED((N, HD), jnp.float32),
    ],
    compiler_params=pltpu.CompilerParams(use_tc_tiling_on_sc=False),
)

R = 400
GRID = N // R


def _mlp1_body(x_ref, p_ref, w1_ref, b1_ref, w2_ref, b2_ref, o_ref, os_ref):
    sgm = x_ref[...] + p_ref[...]
    t = jnp.dot(sgm, w1_ref[...], preferred_element_type=jnp.float32)
    t = jnp.maximum(t + b1_ref[...], 0.0)
    h = jnp.dot(t, w2_ref[...], preferred_element_type=jnp.float32)
    h = jnp.maximum(h + b2_ref[...], 0.0)
    o_ref[...] = h
    os_ref[0] = h[:, :HD]
    os_ref[1] = h[:, HD:]


_mlp1 = pl.pallas_call(
    _mlp1_body,
    grid=(GRID,),
    in_specs=[
        pl.BlockSpec((R, D), lambda i: (i, 0)),
        pl.BlockSpec((R, D), lambda i: (i, 0)),
        pl.BlockSpec((D, D), lambda i: (0, 0)),
        pl.BlockSpec((1, D), lambda i: (0, 0)),
        pl.BlockSpec((D, D), lambda i: (0, 0)),
        pl.BlockSpec((1, D), lambda i: (0, 0)),
    ],
    out_specs=[
        pl.BlockSpec((R, D), lambda i: (i, 0)),
        pl.BlockSpec((NC, R, HD), lambda i: (0, i, 0)),
    ],
    out_shape=[
        jax.ShapeDtypeStruct((N, D), jnp.float32),
        jax.ShapeDtypeStruct((NC, N, HD), jnp.float32),
    ],
)


def _mlp2_body(h_ref, p_ref, w1_ref, b1_ref, w2_ref, b2_ref, cs_ref, o_ref):
    i = pl.program_id(0)
    sgm = h_ref[...] + p_ref[...]
    g = jnp.dot(sgm, w1_ref[...], preferred_element_type=jnp.float32)
    g = jnp.maximum(g + b1_ref[...], 0.0)
    part = jnp.sum(g, axis=0, keepdims=True)

    @pl.when(i == 0)
    def _():
        cs_ref[...] = jnp.zeros_like(cs_ref)

    cs_ref[...] += part

    @pl.when(i == GRID - 1)
    def _():
        v = cs_ref[...] * (1.0 / N)
        o_ref[...] = jnp.dot(v, w2_ref[...],
                             preferred_element_type=jnp.float32) + b2_ref[...]


_mlp2 = pl.pallas_call(
    _mlp2_body,
    grid=(GRID,),
    in_specs=[
        pl.BlockSpec((R, D), lambda i: (i, 0)),
        pl.BlockSpec((R, D), lambda i: (i, 0)),
        pl.BlockSpec((D, D), lambda i: (0, 0)),
        pl.BlockSpec((1, D), lambda i: (0, 0)),
        pl.BlockSpec((D, D), lambda i: (0, 0)),
        pl.BlockSpec((1, D), lambda i: (0, 0)),
    ],
    out_specs=[
        pl.BlockSpec((1, D), lambda i: (0, 0)),
        pl.BlockSpec((1, D), lambda i: (0, 0)),
    ],
    out_shape=[
        jax.ShapeDtypeStruct((1, D), jnp.float32),
        jax.ShapeDtypeStruct((1, D), jnp.float32),
    ],
)


def kernel(x, edge_index, W1a, b1a, W2a, b2a, W1b, b1b, W2b, b2b, batch_size):
    src_r = edge_index[0].reshape(NS, NCHUNK, CHUNK)
    dst_r = edge_index[1].reshape(NS, NCHUNK, CHUNK)
    zeros = jnp.zeros((N, HD), jnp.float32)
    b1a_, b2a_, b1b_, b2b_ = (b.reshape(1, D) for b in (b1a, b2a, b1b, b2b))

    xs = jnp.stack([x[:, :HD], x[:, HD:]], axis=0)

    p1 = _agg(xs, src_r, dst_r, zeros)
    h, hs = _mlp1(x, p1, W1a, b1a_, W2a, b2a_)
    p2 = _agg(hs, src_r, dst_r, zeros)
    _, out = _mlp2(h, p2, W1b, b1b_, W2b, b2b_)
    return out.reshape(-1)


# speedup vs baseline: 1.0296x; 1.0296x over previous
import jax
import jax.numpy as jnp
from jax import lax
from jax.experimental import pallas as pl
from jax.experimental.pallas import tpu as pltpu
from jax.experimental.pallas import tpu_sc as plsc

N = 10000
E = 320000
D = 128
HD = D // 2

NC = 2
NS = 16
EPT = E // NS
CHUNK = 80
NCHUNK = EPT // CHUNK
NBUF = 9
NROUND = -(-NCHUNK // NBUF)
RPS = 624
RPS_LAST = N - (NS - 1) * RPS

_mesh = plsc.VectorSubcoreMesh(
    core_axis_name="c", subcore_axis_name="s", num_cores=NC, num_subcores=NS
)


def _agg_body(xs_hbm, src_hbm, dst_hbm, out_hbm,
              src_v, dst_v, rows_v, gsem, ssem, acc_sh):
    c = lax.axis_index("c")
    s = lax.axis_index("s")

    @pl.when(s < NS - 1)
    def _():
        pltpu.sync_copy(xs_hbm.at[c, pl.ds(s * RPS, RPS)],
                        acc_sh.at[pl.ds(s * RPS, RPS)])

    @pl.when(s == NS - 1)
    def _():
        pltpu.sync_copy(xs_hbm.at[c, pl.ds((NS - 1) * RPS, RPS_LAST)],
                        acc_sh.at[pl.ds((NS - 1) * RPS, RPS_LAST)])

    pltpu.sync_copy(src_hbm.at[s], src_v)
    pltpu.sync_copy(dst_hbm.at[s], dst_v)
    plsc.subcore_barrier()

    xc = xs_hbm.at[c]

    for b in range(NBUF):
        pltpu.async_copy(xc.at[src_v.at[b]], rows_v.at[b], gsem.at[b])

    def round_body(g, cc):
        for b in range(NBUF):
            j = g * NBUF + b

            @pl.when(j < NCHUNK)
            def _():
                pltpu.make_async_copy(xc.at[src_v.at[j]], rows_v.at[b],
                                      gsem.at[b]).wait()
                pltpu.async_copy(rows_v.at[b], acc_sh.at[dst_v.at[j]],
                                 ssem.at[b], add=True)
        for b in range(NBUF):
            jn = (g + 1) * NBUF + b

            @pl.when(jn < NCHUNK)
            def _():
                pltpu.make_async_copy(rows_v.at[b], acc_sh.at[dst_v.at[jn]],
                                      ssem.at[b]).wait()
                pltpu.async_copy(xc.at[src_v.at[jn]], rows_v.at[b],
                                 gsem.at[b])
        return cc

    lax.fori_loop(0, NROUND, round_body, 0)
    for b in range(NBUF):
        pltpu.make_async_copy(rows_v.at[b], acc_sh.at[dst_v.at[b]],
                              ssem.at[b]).wait()
    plsc.subcore_barrier()

    @pl.when(s < NS - 1)
    def _():
        pltpu.sync_copy(acc_sh.at[pl.ds(s * RPS, RPS)],
                        out_hbm.at[pl.ds(s * RPS, RPS), pl.ds(c * HD, HD)])

    @pl.when(s == NS - 1)
    def _():
        pltpu.sync_copy(acc_sh.at[pl.ds((NS - 1) * RPS, RPS_LAST)],
                        out_hbm.at[pl.ds((NS - 1) * RPS, RPS_LAST),
                                   pl.ds(c * HD, HD)])


_agg = pl.kernel(
    _agg_body,
    out_type=jax.ShapeDtypeStruct((N, D), jnp.float32),
    mesh=_mesh,
    scratch_types=[
        pltpu.VMEM((NCHUNK, CHUNK), jnp.int32),
        pltpu.VMEM((NCHUNK, CHUNK), jnp.int32),
        pltpu.VMEM((NBUF, CHUNK, HD), jnp.float32),
        pltpu.SemaphoreType.DMA((NBUF,)),
        pltpu.SemaphoreType.DMA((NBUF,)),
        pltpu.VMEM_SHARED((N, HD), jnp.float32),
    ],
    compiler_params=pltpu.CompilerParams(use_tc_tiling_on_sc=False),
)

R = 400
GRID = N // R


def _mlp1_body(p_ref, w1_ref, b1_ref, w2_ref, b2_ref, os_ref):
    sgm = p_ref[...]
    t = jnp.dot(sgm, w1_ref[...], preferred_element_type=jnp.float32)
    t = jnp.maximum(t + b1_ref[...], 0.0)
    h = jnp.dot(t, w2_ref[...], preferred_element_type=jnp.float32)
    h = jnp.maximum(h + b2_ref[...], 0.0)
    os_ref[0] = h[:, :HD]
    os_ref[1] = h[:, HD:]


_mlp1 = pl.pallas_call(
    _mlp1_body,
    grid=(GRID,),
    in_specs=[
        pl.BlockSpec((R, D), lambda i: (i, 0)),
        pl.BlockSpec((D, D), lambda i: (0, 0)),
        pl.BlockSpec((1, D), lambda i: (0, 0)),
        pl.BlockSpec((D, D), lambda i: (0, 0)),
        pl.BlockSpec((1, D), lambda i: (0, 0)),
    ],
    out_specs=pl.BlockSpec((NC, R, HD), lambda i: (0, i, 0)),
    out_shape=jax.ShapeDtypeStruct((NC, N, HD), jnp.float32),
)


def _mlp2_body(p_ref, w1_ref, b1_ref, w2_ref, b2_ref, cs_ref, o_ref):
    i = pl.program_id(0)
    sgm = p_ref[...]
    g = jnp.dot(sgm, w1_ref[...], preferred_element_type=jnp.float32)
    g = jnp.maximum(g + b1_ref[...], 0.0)
    part = jnp.sum(g, axis=0, keepdims=True)

    @pl.when(i == 0)
    def _():
        cs_ref[...] = jnp.zeros_like(cs_ref)

    cs_ref[...] += part

    @pl.when(i == GRID - 1)
    def _():
        v = cs_ref[...] * (1.0 / N)
        o_ref[...] = jnp.dot(v, w2_ref[...],
                             preferred_element_type=jnp.float32) + b2_ref[...]


_mlp2 = pl.pallas_call(
    _mlp2_body,
    grid=(GRID,),
    in_specs=[
        pl.BlockSpec((R, D), lambda i: (i, 0)),
        pl.BlockSpec((D, D), lambda i: (0, 0)),
        pl.BlockSpec((1, D), lambda i: (0, 0)),
        pl.BlockSpec((D, D), lambda i: (0, 0)),
        pl.BlockSpec((1, D), lambda i: (0, 0)),
    ],
    out_specs=[
        pl.BlockSpec((1, D), lambda i: (0, 0)),
        pl.BlockSpec((1, D), lambda i: (0, 0)),
    ],
    out_shape=[
        jax.ShapeDtypeStruct((1, D), jnp.float32),
        jax.ShapeDtypeStruct((1, D), jnp.float32),
    ],
)


def kernel(x, edge_index, W1a, b1a, W2a, b2a, W1b, b1b, W2b, b2b, batch_size):
    src_r = edge_index[0].reshape(NS, NCHUNK, CHUNK)
    dst_r = edge_index[1].reshape(NS, NCHUNK, CHUNK)
    b1a_, b2a_, b1b_, b2b_ = (b.reshape(1, D) for b in (b1a, b2a, b1b, b2b))
    xs = jnp.stack([x[:, :HD], x[:, HD:]], axis=0)

    p1 = _agg(xs, src_r, dst_r)
    hs = _mlp1(p1, W1a, b1a_, W2a, b2a_)
    p2 = _agg(hs, src_r, dst_r)
    _, out = _mlp2(p2, W1b, b1b_, W2b, b2b_)
    return out.reshape(-1)


# speedup vs baseline: 1.0867x; 1.0555x over previous
import jax
import jax.numpy as jnp
from jax import lax
from jax.experimental import pallas as pl
from jax.experimental.pallas import tpu as pltpu
from jax.experimental.pallas import tpu_sc as plsc

N = 10000
E = 320000
D = 128
HD = D // 2

NC = 2
NS = 16
EPT = E // NS
CHUNK = 80
NCHUNK = EPT // CHUNK
NBUF = 9
NROUND = -(-NCHUNK // NBUF)
RPS = 624
RPS_LAST = N - (NS - 1) * RPS

_mesh = plsc.VectorSubcoreMesh(
    core_axis_name="c", subcore_axis_name="s", num_cores=NC, num_subcores=NS
)


def _agg_body(xs_hbm, src_hbm, dst_hbm, out_hbm,
              src_v, dst_v, rows_v, gsem, ssem, acc_sh):
    c = lax.axis_index("c")
    s = lax.axis_index("s")

    @pl.when(s < NS - 1)
    def _():
        pltpu.sync_copy(xs_hbm.at[c, pl.ds(s * RPS, RPS)],
                        acc_sh.at[pl.ds(s * RPS, RPS)])

    @pl.when(s == NS - 1)
    def _():
        pltpu.sync_copy(xs_hbm.at[c, pl.ds((NS - 1) * RPS, RPS_LAST)],
                        acc_sh.at[pl.ds((NS - 1) * RPS, RPS_LAST)])

    pltpu.sync_copy(src_hbm.at[s], src_v)
    pltpu.sync_copy(dst_hbm.at[s], dst_v)
    plsc.subcore_barrier()

    xc = xs_hbm.at[c]

    for b in range(NBUF):
        pltpu.async_copy(xc.at[src_v.at[b]], rows_v.at[b], gsem.at[b])

    def round_body(g, cc):
        for b in range(NBUF):
            j = g * NBUF + b

            @pl.when(j < NCHUNK)
            def _():
                pltpu.make_async_copy(xc.at[src_v.at[j]], rows_v.at[b],
                                      gsem.at[b]).wait()
                pltpu.async_copy(rows_v.at[b], acc_sh.at[dst_v.at[j]],
                                 ssem.at[b], add=True)
        for b in range(NBUF):
            jn = (g + 1) * NBUF + b

            @pl.when(jn < NCHUNK)
            def _():
                pltpu.make_async_copy(rows_v.at[b], acc_sh.at[dst_v.at[jn]],
                                      ssem.at[b]).wait()
                pltpu.async_copy(xc.at[src_v.at[jn]], rows_v.at[b],
                                 gsem.at[b])
        return cc

    lax.fori_loop(0, NROUND, round_body, 0)
    for b in range(NBUF):
        pltpu.make_async_copy(rows_v.at[b], acc_sh.at[dst_v.at[b]],
                              ssem.at[b]).wait()
    plsc.subcore_barrier()

    @pl.when(s < NS - 1)
    def _():
        pltpu.sync_copy(acc_sh.at[pl.ds(s * RPS, RPS)],
                        out_hbm.at[pl.ds(s * RPS, RPS), pl.ds(c * HD, HD)])

    @pl.when(s == NS - 1)
    def _():
        pltpu.sync_copy(acc_sh.at[pl.ds((NS - 1) * RPS, RPS_LAST)],
                        out_hbm.at[pl.ds((NS - 1) * RPS, RPS_LAST),
                                   pl.ds(c * HD, HD)])


_agg = pl.kernel(
    _agg_body,
    out_type=jax.ShapeDtypeStruct((N, D), jnp.float32),
    mesh=_mesh,
    scratch_types=[
        pltpu.VMEM((NCHUNK, CHUNK), jnp.int32),
        pltpu.VMEM((NCHUNK, CHUNK), jnp.int32),
        pltpu.VMEM((NBUF, CHUNK, HD), jnp.float32),
        pltpu.SemaphoreType.DMA((NBUF,)),
        pltpu.SemaphoreType.DMA((NBUF,)),
        pltpu.VMEM_SHARED((N, HD), jnp.float32),
    ],
    compiler_params=pltpu.CompilerParams(use_tc_tiling_on_sc=False),
)

R = 1000
GRID = N // R


def _mlp1_body(p_ref, w1_ref, b1_ref, w2_ref, b2_ref, os_ref):
    sgm = p_ref[...]
    t = jnp.dot(sgm, w1_ref[...], preferred_element_type=jnp.float32)
    t = jnp.maximum(t + b1_ref[...], 0.0)
    h = jnp.dot(t, w2_ref[...], preferred_element_type=jnp.float32)
    h = jnp.maximum(h + b2_ref[...], 0.0)
    os_ref[0] = h[:, :HD]
    os_ref[1] = h[:, HD:]


_mlp1 = pl.pallas_call(
    _mlp1_body,
    grid=(GRID,),
    in_specs=[
        pl.BlockSpec((R, D), lambda i: (i, 0)),
        pl.BlockSpec((D, D), lambda i: (0, 0)),
        pl.BlockSpec((1, D), lambda i: (0, 0)),
        pl.BlockSpec((D, D), lambda i: (0, 0)),
        pl.BlockSpec((1, D), lambda i: (0, 0)),
    ],
    out_specs=pl.BlockSpec((NC, R, HD), lambda i: (0, i, 0)),
    out_shape=jax.ShapeDtypeStruct((NC, N, HD), jnp.float32),
)


def _mlp2_body(p_ref, w1_ref, b1_ref, w2_ref, b2_ref, cs_ref, o_ref):
    i = pl.program_id(0)
    sgm = p_ref[...]
    g = jnp.dot(sgm, w1_ref[...], preferred_element_type=jnp.float32)
    g = jnp.maximum(g + b1_ref[...], 0.0)
    part = jnp.sum(g, axis=0, keepdims=True)

    @pl.when(i == 0)
    def _():
        cs_ref[...] = jnp.zeros_like(cs_ref)

    cs_ref[...] += part

    @pl.when(i == GRID - 1)
    def _():
        v = cs_ref[...] * (1.0 / N)
        o_ref[...] = jnp.dot(v, w2_ref[...],
                             preferred_element_type=jnp.float32) + b2_ref[...]


_mlp2 = pl.pallas_call(
    _mlp2_body,
    grid=(GRID,),
    in_specs=[
        pl.BlockSpec((R, D), lambda i: (i, 0)),
        pl.BlockSpec((D, D), lambda i: (0, 0)),
        pl.BlockSpec((1, D), lambda i: (0, 0)),
        pl.BlockSpec((D, D), lambda i: (0, 0)),
        pl.BlockSpec((1, D), lambda i: (0, 0)),
    ],
    out_specs=[
        pl.BlockSpec((1, D), lambda i: (0, 0)),
        pl.BlockSpec((1, D), lambda i: (0, 0)),
    ],
    out_shape=[
        jax.ShapeDtypeStruct((1, D), jnp.float32),
        jax.ShapeDtypeStruct((1, D), jnp.float32),
    ],
)


def kernel(x, edge_index, W1a, b1a, W2a, b2a, W1b, b1b, W2b, b2b, batch_size):
    src_r = edge_index[0].reshape(NS, NCHUNK, CHUNK)
    dst_r = edge_index[1].reshape(NS, NCHUNK, CHUNK)
    b1a_, b2a_, b1b_, b2b_ = (b.reshape(1, D) for b in (b1a, b2a, b1b, b2b))
    xs = jnp.stack([x[:, :HD], x[:, HD:]], axis=0)

    p1 = _agg(xs, src_r, dst_r)
    hs = _mlp1(p1, W1a, b1a_, W2a, b2a_)
    p2 = _agg(hs, src_r, dst_r)
    _, out = _mlp2(p2, W1b, b1b_, W2b, b2b_)
    return out.reshape(-1)


# speedup vs baseline: 1.1238x; 1.0341x over previous
import jax
import jax.numpy as jnp
from jax import lax
from jax.experimental import pallas as pl
from jax.experimental.pallas import tpu as pltpu
from jax.experimental.pallas import tpu_sc as plsc

N = 10000
E = 320000
D = 128
HD = D // 2

NC = 2
NS = 16
EPT = E // NS
CHUNK = 80
NCHUNK = EPT // CHUNK
NBUF = 9
NROUND = -(-NCHUNK // NBUF)
RPS = 624
RPS_LAST = N - (NS - 1) * RPS

_mesh = plsc.VectorSubcoreMesh(
    core_axis_name="c", subcore_axis_name="s", num_cores=NC, num_subcores=NS
)


def _agg_body(xs_hbm, src_hbm, dst_hbm, out_hbm,
              src_v, dst_v, rows_v, gsem, ssem, acc_sh):
    c = lax.axis_index("c")
    s = lax.axis_index("s")

    @pl.when(s < NS - 1)
    def _():
        pltpu.async_copy(xs_hbm.at[c, pl.ds(s * RPS, RPS)],
                         acc_sh.at[pl.ds(s * RPS, RPS)], gsem.at[0])

    @pl.when(s == NS - 1)
    def _():
        pltpu.async_copy(xs_hbm.at[c, pl.ds((NS - 1) * RPS, RPS_LAST)],
                         acc_sh.at[pl.ds((NS - 1) * RPS, RPS_LAST)],
                         gsem.at[0])

    pltpu.async_copy(src_hbm.at[s], src_v, gsem.at[1])
    pltpu.async_copy(dst_hbm.at[s], dst_v, gsem.at[2])

    @pl.when(s < NS - 1)
    def _():
        pltpu.make_async_copy(xs_hbm.at[c, pl.ds(s * RPS, RPS)],
                              acc_sh.at[pl.ds(s * RPS, RPS)],
                              gsem.at[0]).wait()

    @pl.when(s == NS - 1)
    def _():
        pltpu.make_async_copy(xs_hbm.at[c, pl.ds((NS - 1) * RPS, RPS_LAST)],
                              acc_sh.at[pl.ds((NS - 1) * RPS, RPS_LAST)],
                              gsem.at[0]).wait()

    pltpu.make_async_copy(src_hbm.at[s], src_v, gsem.at[1]).wait()
    pltpu.make_async_copy(dst_hbm.at[s], dst_v, gsem.at[2]).wait()
    plsc.subcore_barrier()

    xc = xs_hbm.at[c]

    for b in range(NBUF):
        pltpu.async_copy(xc.at[src_v.at[b]], rows_v.at[b], gsem.at[b])

    def round_body(g, cc):
        for b in range(NBUF):
            j = g * NBUF + b

            @pl.when(j < NCHUNK)
            def _():
                pltpu.make_async_copy(xc.at[src_v.at[j]], rows_v.at[b],
                                      gsem.at[b]).wait()
                pltpu.async_copy(rows_v.at[b], acc_sh.at[dst_v.at[j]],
                                 ssem.at[b], add=True)
        for b in range(NBUF):
            jn = (g + 1) * NBUF + b

            @pl.when(jn < NCHUNK)
            def _():
                pltpu.make_async_copy(rows_v.at[b], acc_sh.at[dst_v.at[jn]],
                                      ssem.at[b]).wait()
                pltpu.async_copy(xc.at[src_v.at[jn]], rows_v.at[b],
                                 gsem.at[b])
        return cc

    lax.fori_loop(0, NROUND, round_body, 0)
    for b in range(NBUF):
        pltpu.make_async_copy(rows_v.at[b], acc_sh.at[dst_v.at[b]],
                              ssem.at[b]).wait()
    plsc.subcore_barrier()

    @pl.when(s < NS - 1)
    def _():
        pltpu.sync_copy(acc_sh.at[pl.ds(s * RPS, RPS)],
                        out_hbm.at[pl.ds(s * RPS, RPS), pl.ds(c * HD, HD)])

    @pl.when(s == NS - 1)
    def _():
        pltpu.sync_copy(acc_sh.at[pl.ds((NS - 1) * RPS, RPS_LAST)],
                        out_hbm.at[pl.ds((NS - 1) * RPS, RPS_LAST),
                                   pl.ds(c * HD, HD)])


_agg = pl.kernel(
    _agg_body,
    out_type=jax.ShapeDtypeStruct((N, D), jnp.float32),
    mesh=_mesh,
    scratch_types=[
        pltpu.VMEM((NCHUNK, CHUNK), jnp.int32),
        pltpu.VMEM((NCHUNK, CHUNK), jnp.int32),
        pltpu.VMEM((NBUF, CHUNK, HD), jnp.float32),
        pltpu.SemaphoreType.DMA((NBUF,)),
        pltpu.SemaphoreType.DMA((NBUF,)),
        pltpu.VMEM_SHARED((N, HD), jnp.float32),
    ],
    compiler_params=pltpu.CompilerParams(use_tc_tiling_on_sc=False),
)

R = 2000
GRID = N // R


def _mlp1_body(p_ref, w1_ref, b1_ref, w2_ref, b2_ref, os_ref):
    sgm = p_ref[...]
    t = jnp.dot(sgm, w1_ref[...], preferred_element_type=jnp.float32)
    t = jnp.maximum(t + b1_ref[...], 0.0)
    h = jnp.dot(t, w2_ref[...], preferred_element_type=jnp.float32)
    h = jnp.maximum(h + b2_ref[...], 0.0)
    os_ref[0] = h[:, :HD]
    os_ref[1] = h[:, HD:]


_mlp1 = pl.pallas_call(
    _mlp1_body,
    grid=(GRID,),
    in_specs=[
        pl.BlockSpec((R, D), lambda i: (i, 0)),
        pl.BlockSpec((D, D), lambda i: (0, 0)),
        pl.BlockSpec((1, D), lambda i: (0, 0)),
        pl.BlockSpec((D, D), lambda i: (0, 0)),
        pl.BlockSpec((1, D), lambda i: (0, 0)),
    ],
    out_specs=pl.BlockSpec((NC, R, HD), lambda i: (0, i, 0)),
    out_shape=jax.ShapeDtypeStruct((NC, N, HD), jnp.float32),
)


def _mlp2_body(p_ref, w1_ref, b1_ref, w2_ref, b2_ref, cs_ref, o_ref):
    i = pl.program_id(0)
    sgm = p_ref[...]
    g = jnp.dot(sgm, w1_ref[...], preferred_element_type=jnp.float32)
    g = jnp.maximum(g + b1_ref[...], 0.0)
    part = jnp.sum(g, axis=0, keepdims=True)

    @pl.when(i == 0)
    def _():
        cs_ref[...] = jnp.zeros_like(cs_ref)

    cs_ref[...] += part

    @pl.when(i == GRID - 1)
    def _():
        v = cs_ref[...] * (1.0 / N)
        o_ref[...] = jnp.dot(v, w2_ref[...],
                             preferred_element_type=jnp.float32) + b2_ref[...]


_mlp2 = pl.pallas_call(
    _mlp2_body,
    grid=(GRID,),
    in_specs=[
        pl.BlockSpec((R, D), lambda i: (i, 0)),
        pl.BlockSpec((D, D), lambda i: (0, 0)),
        pl.BlockSpec((1, D), lambda i: (0, 0)),
        pl.BlockSpec((D, D), lambda i: (0, 0)),
        pl.BlockSpec((1, D), lambda i: (0, 0)),
    ],
    out_specs=[
        pl.BlockSpec((1, D), lambda i: (0, 0)),
        pl.BlockSpec((1, D), lambda i: (0, 0)),
    ],
    out_shape=[
        jax.ShapeDtypeStruct((1, D), jnp.float32),
        jax.ShapeDtypeStruct((1, D), jnp.float32),
    ],
)


def kernel(x, edge_index, W1a, b1a, W2a, b2a, W1b, b1b, W2b, b2b, batch_size):
    src_r = edge_index[0].reshape(NS, NCHUNK, CHUNK)
    dst_r = edge_index[1].reshape(NS, NCHUNK, CHUNK)
    b1a_, b2a_, b1b_, b2b_ = (b.reshape(1, D) for b in (b1a, b2a, b1b, b2b))
    xs = jnp.stack([x[:, :HD], x[:, HD:]], axis=0)

    p1 = _agg(xs, src_r, dst_r)
    hs = _mlp1(p1, W1a, b1a_, W2a, b2a_)
    p2 = _agg(hs, src_r, dst_r)
    _, out = _mlp2(p2, W1b, b1b_, W2b, b2b_)
    return out.reshape(-1)
